# baseline (speedup 1.0000x reference)
import functools

import jax
import jax.numpy as jnp
from jax.experimental import pallas as pl
from jax.experimental.pallas import tpu as pltpu
from jax.experimental.pallas import tpu_sc as plsc

B = 8
L = 2048
D = 128
WINDOW = 9
K_SYM = 5
PATCH_HIGH = 3
PATCH_LOW = 12
ENT_THR_ROBUST = 1.51
NCAND = (L + PATCH_HIGH - 1) // PATCH_HIGH
KP = 704
CSLEN = 2112


def _ent_body(x_ref, ent_ref, cs_ref):
    x = x_ref[...]
    z4 = jnp.zeros((B, WINDOW // 2), jnp.float32)
    counts = []
    for s in range(K_SYM):
        ind = jnp.concatenate(
            [z4, (x == s).astype(jnp.float32), z4], axis=1)
        c = ind[:, 4:4 + L]
        for w in range(WINDOW):
            if w != 4:
                c = c + ind[:, w:w + L]
        counts.append(c)
    total = counts[0] + counts[1] + counts[2] + counts[3] + counts[4]
    total = jnp.maximum(total, 1e-12)
    ent = jnp.zeros((B, L), jnp.float32)
    for s in range(K_SYM):
        p = counts[s] / total
        ent = ent - p * jnp.log2(p + 1e-12)
    ent_ref[...] = ent

    xf = x.astype(jnp.float32)
    inc = xf
    sh = 1
    while sh < L:
        z = jnp.zeros((B, sh), jnp.float32)
        inc = inc + jnp.concatenate([z, inc[:, :L - sh]], axis=1)
        sh *= 2
    zcol = jnp.zeros((B, 1), jnp.float32)
    ztail = jnp.zeros((B, CSLEN - L - 1), jnp.float32)
    cs_ref[...] = jnp.concatenate([zcol, inc, ztail], axis=1)


def _entropy_cs(x):
    return pl.pallas_call(
        _ent_body,
        out_shape=[
            jax.ShapeDtypeStruct((B, L), jnp.float32),
            jax.ShapeDtypeStruct((B, CSLEN), jnp.float32),
        ],
    )(x)


def _walk_patches(entropy, cs):
    mesh = plsc.VectorSubcoreMesh(core_axis_name="c", subcore_axis_name="s",
                                  num_cores=1)
    nworkers = 16

    @functools.partial(
        pl.kernel,
        out_type=[
            jax.ShapeDtypeStruct((B, KP), jnp.float32),
            jax.ShapeDtypeStruct((B, 16), jnp.float32),
        ],
        mesh=mesh,
        scratch_types=[
            pltpu.VMEM((L + 64,), jnp.float32),
            pltpu.VMEM((CSLEN,), jnp.float32),
            pltpu.VMEM((KP,), jnp.float32),
            pltpu.VMEM((16,), jnp.float32),
        ],
    )
    def walk(ent_hbm, cs_hbm, pm_hbm, cnt_hbm, ent_v, cs_v, pm_v, cnt_v):
        wid = jax.lax.axis_index("s") + jax.lax.axis_index("c")
        row = jax.lax.rem(wid, B)
        pltpu.sync_copy(ent_hbm.at[row], ent_v.at[pl.ds(0, L)])
        pltpu.sync_copy(cs_hbm.at[row], cs_v)
        zero16 = jnp.zeros((16,), jnp.float32)
        ent_v[pl.ds(L, 16)] = zero16
        ent_v[pl.ds(L + 16, 16)] = zero16
        ent_v[pl.ds(L + 32, 16)] = zero16
        ent_v[pl.ds(L + 48, 16)] = zero16
        lane_i = jax.lax.iota(jnp.int32, 16)
        lane0_f = (1 - jnp.minimum(lane_i, 1)).astype(jnp.float32)
        d1 = lane_i - 1
        lane1_f = (1 - jnp.minimum(d1 * d1, 1)).astype(jnp.float32)

        def body(_, carry):
            i, t, last, e_cur, cs_cur = carry
            act_a = i < L
            act_af = jnp.where(act_a, 1.0, 0.0)
            hi_a = e_cur > ENT_THR_ROBUST
            eA3 = ent_v[pl.ds(i + 3, 16)][0]
            eA12 = ent_v[pl.ds(i + 12, 16)][0]
            cA3 = cs_v[pl.ds(i + 3, 16)][0]
            cA12 = cs_v[pl.ds(i + 12, 16)][0]
            eB6 = ent_v[pl.ds(i + 6, 16)][0]
            eB15 = ent_v[pl.ds(i + 15, 16)][0]
            eB24 = ent_v[pl.ds(i + 24, 16)][0]
            cB6 = cs_v[pl.ds(i + 6, 16)][0]
            cB15 = cs_v[pl.ds(i + 15, 16)][0]
            cB24 = cs_v[pl.ds(i + 24, 16)][0]
            csA = jnp.where(hi_a, cA3, cA12)
            rdenA = jnp.where(hi_a, 1.0 / PATCH_HIGH, 1.0 / PATCH_LOW)
            pmA = (csA - cs_cur) * rdenA
            iA = jnp.where(act_a, i + jnp.where(hi_a, PATCH_HIGH, PATCH_LOW),
                           i)
            lastA = jnp.where(act_a, i, last)
            eA = jnp.where(hi_a, eA3, eA12)
            tA = t + jnp.where(act_a, 1, 0)
            act_b = iA < L
            hi_b = eA > ENT_THR_ROBUST
            csB = jnp.where(hi_b, jnp.where(hi_a, cB6, cB15),
                            jnp.where(hi_a, cB15, cB24))
            eB = jnp.where(hi_b, jnp.where(hi_a, eB6, eB15),
                           jnp.where(hi_a, eB15, eB24))
            rdenB = jnp.where(hi_b, 1.0 / PATCH_HIGH, 1.0 / PATCH_LOW)
            pmB = (csB - csA) * rdenB
            iB = jnp.where(act_b,
                           iA + jnp.where(hi_b, PATCH_HIGH, PATCH_LOW), iA)
            lastB = jnp.where(act_b, iA, lastA)
            tB = tA + jnp.where(act_b, 1, 0)
            laneB_f = lane1_f * act_af + lane0_f * (1.0 - act_af)
            pm_v[pl.ds(t, 16)] = lane0_f * pmA + laneB_f * pmB
            return (iB, tB, lastB, eB, csB)

        e0 = ent_v[pl.ds(0, 16)][0]
        init = (jnp.int32(0), jnp.int32(0), jnp.int32(0), e0,
                jnp.float32(0.0))
        final = jax.lax.fori_loop(0, NCAND // 2 + 1, body, init, unroll=8)
        count = final[1]
        i_last = final[2]
        cs_total = cs_v[pl.ds(L, 16)][0]
        cs_il = cs_v[pl.ds(i_last, 16)][0]
        def lane(k):
            dk = lane_i - k
            return (1 - jnp.minimum(dk * dk, 1)).astype(jnp.float32)
        cnt_v[...] = (lane(0) * count.astype(jnp.float32)
                      + lane(1) * i_last.astype(jnp.float32)
                      + lane(2) * cs_total + lane(3) * cs_il)

        @pl.when(wid < B)
        def _():
            pltpu.sync_copy(pm_v, pm_hbm.at[wid])
            pltpu.sync_copy(cnt_v, cnt_hbm.at[wid])

    return walk(entropy, cs)


def _feat_body(pm_ref, cnt_ref, w1_ref, b1_ref, w2_ref, b2_ref, out_ref):
    pm = pm_ref[...]
    count = cnt_ref[:, 0:1]
    i_last = cnt_ref[:, 1:2]
    cs_total = cnt_ref[:, 2:3]
    cs_il = cnt_ref[:, 3:4]
    den_last = jnp.maximum(float(L) - i_last, 1.0)
    pm_last = (cs_total - cs_il) / den_last
    tt = jax.lax.broadcasted_iota(jnp.int32, (B, KP), 1).astype(jnp.float32)
    pm = jnp.where(tt == count - 1.0, pm_last, pm)
    msk = (tt < count).astype(jnp.float32)
    w1 = w1_ref[...]
    b1 = b1_ref[...]
    h = jnp.maximum(pm[:, :, None] * w1 + b1, 0.0) * msk[:, :, None]
    s_h = jnp.sum(h, axis=1)
    out = jax.lax.dot_general(
        s_h, w2_ref[...], (((1,), (0,)), ((), ())),
        preferred_element_type=jnp.float32,
    )
    out_ref[...] = out / count + b2_ref[...]


def _features(pm, cnt, W1, b1, W2, b2):
    return pl.pallas_call(
        _feat_body,
        out_shape=jax.ShapeDtypeStruct((B, D), jnp.float32),
    )(pm, cnt, W1, b1.reshape(1, D), W2, b2.reshape(1, D))


def kernel(x, W1, b1, W2, b2):
    entropy, cs = _entropy_cs(x)
    pm, cnt = _walk_patches(entropy, cs)
    blt = _features(pm, cnt, W1, b1, W2, b2)
    return (blt, entropy)


# speedup vs baseline: 1.0244x; 1.0244x over previous
import functools

import jax
import jax.numpy as jnp
from jax.experimental import pallas as pl
from jax.experimental.pallas import tpu as pltpu
from jax.experimental.pallas import tpu_sc as plsc

B = 8
L = 2048
D = 128
WINDOW = 9
K_SYM = 5
PATCH_HIGH = 3
PATCH_LOW = 12
ENT_THR_ROBUST = 1.51
NCAND = (L + PATCH_HIGH - 1) // PATCH_HIGH
KP = 704
CSLEN = 2112


def _ent_body(x_ref, ent_ref, cs_ref):
    x = x_ref[...]
    z4 = jnp.zeros((B, WINDOW // 2), jnp.float32)
    counts = []
    for s in range(K_SYM):
        ind = jnp.concatenate(
            [z4, (x == s).astype(jnp.float32), z4], axis=1)
        c = ind[:, 4:4 + L]
        for w in range(WINDOW):
            if w != 4:
                c = c + ind[:, w:w + L]
        counts.append(c)
    total = counts[0] + counts[1] + counts[2] + counts[3] + counts[4]
    total = jnp.maximum(total, 1e-12)
    ent = jnp.zeros((B, L), jnp.float32)
    for s in range(K_SYM):
        p = counts[s] / total
        ent = ent - p * jnp.log2(p + 1e-12)
    ent_ref[...] = ent

    xf = x.astype(jnp.float32)
    inc = xf
    sh = 1
    while sh < L:
        z = jnp.zeros((B, sh), jnp.float32)
        inc = inc + jnp.concatenate([z, inc[:, :L - sh]], axis=1)
        sh *= 2
    zcol = jnp.zeros((B, 1), jnp.float32)
    ztail = jnp.zeros((B, CSLEN - L - 1), jnp.float32)
    cs_ref[...] = jnp.concatenate([zcol, inc, ztail], axis=1)


def _entropy_cs(x):
    return pl.pallas_call(
        _ent_body,
        out_shape=[
            jax.ShapeDtypeStruct((B, L), jnp.float32),
            jax.ShapeDtypeStruct((B, CSLEN), jnp.float32),
        ],
    )(x)


def _walk_patches(entropy, cs):
    mesh = plsc.VectorSubcoreMesh(core_axis_name="c", subcore_axis_name="s",
                                  num_cores=1)

    @functools.partial(
        pl.kernel,
        out_type=[
            jax.ShapeDtypeStruct((B, KP), jnp.float32),
            jax.ShapeDtypeStruct((B, 16), jnp.float32),
        ],
        mesh=mesh,
        scratch_types=[
            pltpu.VMEM((L + 64,), jnp.float32),
            pltpu.VMEM((CSLEN,), jnp.float32),
            pltpu.VMEM((KP,), jnp.float32),
            pltpu.VMEM((16,), jnp.float32),
        ],
    )
    def walk(ent_hbm, cs_hbm, pm_hbm, cnt_hbm, ent_v, cs_v, pm_v, cnt_v):
        wid = jax.lax.axis_index("s") + jax.lax.axis_index("c")
        row = jax.lax.rem(wid, B)
        pltpu.sync_copy(ent_hbm.at[row], ent_v.at[pl.ds(0, L)])
        pltpu.sync_copy(cs_hbm.at[row], cs_v)
        zero16 = jnp.zeros((16,), jnp.float32)
        ent_v[pl.ds(L, 16)] = zero16
        ent_v[pl.ds(L + 16, 16)] = zero16
        ent_v[pl.ds(L + 32, 16)] = zero16
        ent_v[pl.ds(L + 48, 16)] = zero16
        lane_i = jax.lax.iota(jnp.int32, 16)
        lane0_f = (1 - jnp.minimum(lane_i, 1)).astype(jnp.float32)
        d1 = lane_i - 1
        lane1_f = (1 - jnp.minimum(d1 * d1, 1)).astype(jnp.float32)

        def body(_, carry):
            i, t, last, e_cur, cs_cur = carry
            act_a = i < L
            act_af = jnp.where(act_a, 1.0, 0.0)
            hi_a = e_cur > ENT_THR_ROBUST
            eA3 = ent_v[pl.ds(i + 3, 16)][0]
            eA12 = ent_v[pl.ds(i + 12, 16)][0]
            cA3 = cs_v[pl.ds(i + 3, 16)][0]
            cA12 = cs_v[pl.ds(i + 12, 16)][0]
            eB6 = ent_v[pl.ds(i + 6, 16)][0]
            eB15 = ent_v[pl.ds(i + 15, 16)][0]
            eB24 = ent_v[pl.ds(i + 24, 16)][0]
            cB6 = cs_v[pl.ds(i + 6, 16)][0]
            cB15 = cs_v[pl.ds(i + 15, 16)][0]
            cB24 = cs_v[pl.ds(i + 24, 16)][0]
            csA = jnp.where(hi_a, cA3, cA12)
            rdenA = jnp.where(hi_a, 1.0 / PATCH_HIGH, 1.0 / PATCH_LOW)
            pmA = (csA - cs_cur) * rdenA
            iA = jnp.where(act_a, i + jnp.where(hi_a, PATCH_HIGH, PATCH_LOW),
                           i)
            lastA = jnp.where(act_a, i, last)
            eA = jnp.where(hi_a, eA3, eA12)
            tA = t + jnp.where(act_a, 1, 0)
            act_b = iA < L
            hi_b = eA > ENT_THR_ROBUST
            csB = jnp.where(hi_b, jnp.where(hi_a, cB6, cB15),
                            jnp.where(hi_a, cB15, cB24))
            eB = jnp.where(hi_b, jnp.where(hi_a, eB6, eB15),
                           jnp.where(hi_a, eB15, eB24))
            rdenB = jnp.where(hi_b, 1.0 / PATCH_HIGH, 1.0 / PATCH_LOW)
            pmB = (csB - csA) * rdenB
            iB = jnp.where(act_b,
                           iA + jnp.where(hi_b, PATCH_HIGH, PATCH_LOW), iA)
            lastB = jnp.where(act_b, iA, lastA)
            tB = tA + jnp.where(act_b, 1, 0)
            laneB_f = lane1_f * act_af + lane0_f * (1.0 - act_af)
            pm_v[pl.ds(t, 16)] = lane0_f * pmA + laneB_f * pmB
            return (iB, tB, lastB, eB, csB)

        e0 = ent_v[pl.ds(0, 16)][0]
        init = (jnp.int32(0), jnp.int32(0), jnp.int32(0), e0,
                jnp.float32(0.0))
        final = jax.lax.fori_loop(0, NCAND // 2 + 1, body, init, unroll=4)
        count = final[1]
        i_last = final[2]
        cs_total = cs_v[pl.ds(L, 16)][0]
        cs_il = cs_v[pl.ds(i_last, 16)][0]
        def lane(k):
            dk = lane_i - k
            return (1 - jnp.minimum(dk * dk, 1)).astype(jnp.float32)
        cnt_v[...] = (lane(0) * count.astype(jnp.float32)
                      + lane(1) * i_last.astype(jnp.float32)
                      + lane(2) * cs_total + lane(3) * cs_il)

        @pl.when(wid < B)
        def _():
            pltpu.sync_copy(pm_v, pm_hbm.at[wid])
            pltpu.sync_copy(cnt_v, cnt_hbm.at[wid])

    return walk(entropy, cs)


def _feat_body(pm_ref, cnt_ref, w1_ref, b1_ref, w2_ref, b2_ref, out_ref):
    pm = pm_ref[...]
    count = cnt_ref[:, 0:1]
    i_last = cnt_ref[:, 1:2]
    cs_total = cnt_ref[:, 2:3]
    cs_il = cnt_ref[:, 3:4]
    den_last = jnp.maximum(float(L) - i_last, 1.0)
    pm_last = (cs_total - cs_il) / den_last
    tt = jax.lax.broadcasted_iota(jnp.int32, (B, KP), 1).astype(jnp.float32)
    pm = jnp.where(tt == count - 1.0, pm_last, pm)
    msk = (tt < count).astype(jnp.float32)
    w1 = w1_ref[...]
    b1 = b1_ref[...]
    h = jnp.maximum(pm[:, :, None] * w1 + b1, 0.0) * msk[:, :, None]
    s_h = jnp.sum(h, axis=1)
    out = jax.lax.dot_general(
        s_h, w2_ref[...], (((1,), (0,)), ((), ())),
        preferred_element_type=jnp.float32,
    )
    out_ref[...] = out / count + b2_ref[...]


def _features(pm, cnt, W1, b1, W2, b2):
    return pl.pallas_call(
        _feat_body,
        out_shape=jax.ShapeDtypeStruct((B, D), jnp.float32),
    )(pm, cnt, W1, b1.reshape(1, D), W2, b2.reshape(1, D))


def kernel(x, W1, b1, W2, b2):
    entropy, cs = _entropy_cs(x)
    pm, cnt = _walk_patches(entropy, cs)
    blt = _features(pm, cnt, W1, b1, W2, b2)
    return (blt, entropy)


# speedup vs baseline: 1.0466x; 1.0216x over previous
import functools

import jax
import jax.numpy as jnp
from jax.experimental import pallas as pl
from jax.experimental.pallas import tpu as pltpu
from jax.experimental.pallas import tpu_sc as plsc

B = 8
L = 2048
D = 128
WINDOW = 9
K_SYM = 5
PATCH_HIGH = 3
PATCH_LOW = 12
ENT_THR_ROBUST = 1.51
NCAND = (L + PATCH_HIGH - 1) // PATCH_HIGH
KP = 704
CSLEN = 2112


def _ent_body(x_ref, ent_ref, cs_ref):
    x = x_ref[...]
    z4 = jnp.zeros((B, WINDOW // 2), jnp.float32)
    counts = []
    for s in range(K_SYM):
        ind = jnp.concatenate(
            [z4, (x == s).astype(jnp.float32), z4], axis=1)
        c = ind[:, 4:4 + L]
        for w in range(WINDOW):
            if w != 4:
                c = c + ind[:, w:w + L]
        counts.append(c)
    total = counts[0] + counts[1] + counts[2] + counts[3] + counts[4]
    total = jnp.maximum(total, 1e-12)
    ent = jnp.zeros((B, L), jnp.float32)
    for s in range(K_SYM):
        p = counts[s] / total
        ent = ent - p * jnp.log2(p + 1e-12)
    ent_ref[...] = ent

    xf = x.astype(jnp.float32)
    inc = xf
    sh = 1
    while sh < L:
        z = jnp.zeros((B, sh), jnp.float32)
        inc = inc + jnp.concatenate([z, inc[:, :L - sh]], axis=1)
        sh *= 2
    zcol = jnp.zeros((B, 1), jnp.float32)
    ztail = jnp.zeros((B, CSLEN - L - 1), jnp.float32)
    cs_ref[...] = jnp.concatenate([zcol, inc, ztail], axis=1)


def _entropy_cs(x):
    return pl.pallas_call(
        _ent_body,
        out_shape=[
            jax.ShapeDtypeStruct((B, L), jnp.float32),
            jax.ShapeDtypeStruct((B, CSLEN), jnp.float32),
        ],
    )(x)


def _walk_patches(entropy, cs):
    mesh = plsc.VectorSubcoreMesh(core_axis_name="c", subcore_axis_name="s",
                                  num_cores=1)

    @functools.partial(
        pl.kernel,
        out_type=[
            jax.ShapeDtypeStruct((B, KP), jnp.float32),
            jax.ShapeDtypeStruct((B, 16), jnp.float32),
        ],
        mesh=mesh,
        scratch_types=[
            pltpu.VMEM((L + 64,), jnp.float32),
            pltpu.VMEM((CSLEN,), jnp.float32),
            pltpu.VMEM((KP,), jnp.float32),
            pltpu.VMEM((16,), jnp.float32),
            pltpu.SemaphoreType.DMA,
            pltpu.SemaphoreType.DMA,
        ],
    )
    def walk(ent_hbm, cs_hbm, pm_hbm, cnt_hbm, ent_v, cs_v, pm_v, cnt_v,
             sem_e, sem_c):
        wid = jax.lax.axis_index("s") + jax.lax.axis_index("c")
        row = jax.lax.rem(wid, B)
        cp_e = pltpu.async_copy(ent_hbm.at[row], ent_v.at[pl.ds(0, L)], sem_e)
        cp_c = pltpu.async_copy(cs_hbm.at[row], cs_v, sem_c)
        cp_e.wait()
        cp_c.wait()
        zero16 = jnp.zeros((16,), jnp.float32)
        ent_v[pl.ds(L, 16)] = zero16
        ent_v[pl.ds(L + 16, 16)] = zero16
        ent_v[pl.ds(L + 32, 16)] = zero16
        ent_v[pl.ds(L + 48, 16)] = zero16
        lane_i = jax.lax.iota(jnp.int32, 16)
        lane0_f = (1 - jnp.minimum(lane_i, 1)).astype(jnp.float32)
        d1 = lane_i - 1
        lane1_f = (1 - jnp.minimum(d1 * d1, 1)).astype(jnp.float32)

        def body(_, carry):
            i, t, last, e_cur, cs_cur = carry
            act_a = i < L
            act_af = jnp.where(act_a, 1.0, 0.0)
            hi_a = e_cur > ENT_THR_ROBUST
            eA3 = ent_v[pl.ds(i + 3, 16)][0]
            eA12 = ent_v[pl.ds(i + 12, 16)][0]
            cA3 = cs_v[pl.ds(i + 3, 16)][0]
            cA12 = cs_v[pl.ds(i + 12, 16)][0]
            eB6 = ent_v[pl.ds(i + 6, 16)][0]
            eB15 = ent_v[pl.ds(i + 15, 16)][0]
            eB24 = ent_v[pl.ds(i + 24, 16)][0]
            cB6 = cs_v[pl.ds(i + 6, 16)][0]
            cB15 = cs_v[pl.ds(i + 15, 16)][0]
            cB24 = cs_v[pl.ds(i + 24, 16)][0]
            csA = jnp.where(hi_a, cA3, cA12)
            rdenA = jnp.where(hi_a, 1.0 / PATCH_HIGH, 1.0 / PATCH_LOW)
            pmA = (csA - cs_cur) * rdenA
            iA = jnp.where(act_a, i + jnp.where(hi_a, PATCH_HIGH, PATCH_LOW),
                           i)
            lastA = jnp.where(act_a, i, last)
            eA = jnp.where(hi_a, eA3, eA12)
            tA = t + jnp.where(act_a, 1, 0)
            act_b = iA < L
            hi_b = eA > ENT_THR_ROBUST
            csB = jnp.where(hi_b, jnp.where(hi_a, cB6, cB15),
                            jnp.where(hi_a, cB15, cB24))
            eB = jnp.where(hi_b, jnp.where(hi_a, eB6, eB15),
                           jnp.where(hi_a, eB15, eB24))
            rdenB = jnp.where(hi_b, 1.0 / PATCH_HIGH, 1.0 / PATCH_LOW)
            pmB = (csB - csA) * rdenB
            iB = jnp.where(act_b,
                           iA + jnp.where(hi_b, PATCH_HIGH, PATCH_LOW), iA)
            lastB = jnp.where(act_b, iA, lastA)
            tB = tA + jnp.where(act_b, 1, 0)
            laneB_f = lane1_f * act_af + lane0_f * (1.0 - act_af)
            pm_v[pl.ds(t, 16)] = lane0_f * pmA + laneB_f * pmB
            return (iB, tB, lastB, eB, csB)

        e0 = ent_v[pl.ds(0, 16)][0]
        init = (jnp.int32(0), jnp.int32(0), jnp.int32(0), e0,
                jnp.float32(0.0))
        final = jax.lax.fori_loop(0, NCAND // 2 + 1, body, init, unroll=4)
        count = final[1]
        i_last = final[2]
        cs_total = cs_v[pl.ds(L, 16)][0]
        cs_il = cs_v[pl.ds(i_last, 16)][0]
        def lane(k):
            dk = lane_i - k
            return (1 - jnp.minimum(dk * dk, 1)).astype(jnp.float32)
        cnt_v[...] = (lane(0) * count.astype(jnp.float32)
                      + lane(1) * i_last.astype(jnp.float32)
                      + lane(2) * cs_total + lane(3) * cs_il)

        @pl.when(wid < B)
        def _():
            pltpu.sync_copy(pm_v, pm_hbm.at[wid])
            pltpu.sync_copy(cnt_v, cnt_hbm.at[wid])

    return walk(entropy, cs)


def _feat_body(pm_ref, cnt_ref, w1_ref, b1_ref, w2_ref, b2_ref, out_ref):
    pm = pm_ref[...]
    count = cnt_ref[:, 0:1]
    i_last = cnt_ref[:, 1:2]
    cs_total = cnt_ref[:, 2:3]
    cs_il = cnt_ref[:, 3:4]
    den_last = jnp.maximum(float(L) - i_last, 1.0)
    pm_last = (cs_total - cs_il) / den_last
    tt = jax.lax.broadcasted_iota(jnp.int32, (B, KP), 1).astype(jnp.float32)
    pm = jnp.where(tt == count - 1.0, pm_last, pm)
    msk = (tt < count).astype(jnp.float32)
    w1 = w1_ref[...]
    b1 = b1_ref[...]
    h = jnp.maximum(pm[:, :, None] * w1 + b1, 0.0) * msk[:, :, None]
    s_h = jnp.sum(h, axis=1)
    out = jax.lax.dot_general(
        s_h, w2_ref[...], (((1,), (0,)), ((), ())),
        preferred_element_type=jnp.float32,
    )
    out_ref[...] = out / count + b2_ref[...]


def _features(pm, cnt, W1, b1, W2, b2):
    return pl.pallas_call(
        _feat_body,
        out_shape=jax.ShapeDtypeStruct((B, D), jnp.float32),
    )(pm, cnt, W1, b1.reshape(1, D), W2, b2.reshape(1, D))


def kernel(x, W1, b1, W2, b2):
    entropy, cs = _entropy_cs(x)
    pm, cnt = _walk_patches(entropy, cs)
    blt = _features(pm, cnt, W1, b1, W2, b2)
    return (blt, entropy)
